# baseline (speedup 1.0000x reference)
import functools

import jax
import jax.numpy as jnp
from jax.experimental import pallas as pl
from jax.experimental.pallas import tpu as pltpu

_BM = 256


def _body(idx_ref, w_ref, a_ref, x_ref, o_ref):
    w = w_ref[idx_ref[0]]
    acc = jnp.dot(a_ref[0], x_ref[...], preferred_element_type=jnp.float32)
    o_ref[...] = w * acc


@functools.partial(jax.jit, static_argnames=("bm",))
def _conv(x, weight, adjs, idx, bm=_BM):
    n, m, k = adjs.shape
    _, d = x.shape
    idx_arr = jnp.asarray(idx, jnp.int32).reshape((1,))
    grid_spec = pltpu.PrefetchScalarGridSpec(
        num_scalar_prefetch=2,
        grid=(m // bm,),
        in_specs=[
            pl.BlockSpec((1, bm, k), lambda i, idx_ref, w_ref: (idx_ref[0], i, 0)),
            pl.BlockSpec((k, d), lambda i, idx_ref, w_ref: (0, 0)),
        ],
        out_specs=pl.BlockSpec((bm, d), lambda i, idx_ref, w_ref: (i, 0)),
    )
    return pl.pallas_call(
        _body,
        grid_spec=grid_spec,
        out_shape=jax.ShapeDtypeStruct((m, d), jnp.float32),
    )(idx_arr, weight, adjs, x)


def kernel(x, weight, adjs, idx):
    return _conv(x, weight, adjs, idx)


# speedup vs baseline: 1.0620x; 1.0620x over previous
import functools

import jax
import jax.numpy as jnp
from jax.experimental import pallas as pl
from jax.experimental.pallas import tpu as pltpu

_BM = 1024


def _body(idx_ref, w_ref, a_ref, x_ref, o_ref):
    w = w_ref[idx_ref[0]]
    acc = jnp.dot(a_ref[0], x_ref[...], preferred_element_type=jnp.float32)
    o_ref[...] = w * acc


@functools.partial(jax.jit, static_argnames=("bm",))
def _conv(x, weight, adjs, idx, bm=_BM):
    n, m, k = adjs.shape
    _, d = x.shape
    idx_arr = jnp.asarray(idx, jnp.int32).reshape((1,))
    grid_spec = pltpu.PrefetchScalarGridSpec(
        num_scalar_prefetch=2,
        grid=(m // bm,),
        in_specs=[
            pl.BlockSpec((1, bm, k), lambda i, idx_ref, w_ref: (idx_ref[0], i, 0)),
            pl.BlockSpec((k, d), lambda i, idx_ref, w_ref: (0, 0)),
        ],
        out_specs=pl.BlockSpec((bm, d), lambda i, idx_ref, w_ref: (i, 0)),
    )
    return pl.pallas_call(
        _body,
        grid_spec=grid_spec,
        out_shape=jax.ShapeDtypeStruct((m, d), jnp.float32),
    )(idx_arr, weight, adjs, x)


def kernel(x, weight, adjs, idx):
    return _conv(x, weight, adjs, idx)


# speedup vs baseline: 1.1493x; 1.0822x over previous
import functools

import jax
import jax.numpy as jnp
from jax.experimental import pallas as pl
from jax.experimental.pallas import tpu as pltpu

_BM = 512


def _body(idx_ref, w_ref, a_ref, x_ref, o_ref):
    w = w_ref[idx_ref[0]]
    acc = jnp.dot(a_ref[0], x_ref[...], preferred_element_type=jnp.float32)
    o_ref[...] = w * acc


@functools.partial(jax.jit, static_argnames=("bm",))
def _conv(x, weight, adjs, idx, bm=_BM):
    n, m, k = adjs.shape
    _, d = x.shape
    idx_arr = jnp.asarray(idx, jnp.int32).reshape((1,))
    grid_spec = pltpu.PrefetchScalarGridSpec(
        num_scalar_prefetch=2,
        grid=(m // bm,),
        in_specs=[
            pl.BlockSpec((1, bm, k), lambda i, idx_ref, w_ref: (idx_ref[0], i, 0)),
            pl.BlockSpec((k, d), lambda i, idx_ref, w_ref: (0, 0)),
        ],
        out_specs=pl.BlockSpec((bm, d), lambda i, idx_ref, w_ref: (i, 0)),
    )
    return pl.pallas_call(
        _body,
        grid_spec=grid_spec,
        out_shape=jax.ShapeDtypeStruct((m, d), jnp.float32),
    )(idx_arr, weight, adjs, x)


def kernel(x, weight, adjs, idx):
    return _conv(x, weight, adjs, idx)
